# baseline (speedup 1.0000x reference)
import functools

import jax
import jax.numpy as jnp
from jax import lax
from jax.experimental import pallas as pl
from jax.experimental.pallas import tpu as pltpu
from jax.experimental.pallas import tpu_sc as plsc

N = 10000
E = 320000
D = 128
_MAX = 1.0 - 1e-5

NC = 2
NS = 16
NW = NC * NS
EW = E // NW
BA = 80
DP = D // 2
NCA = EW // BA
ET = E // NS
BC = 80
NCC = ET // BC
ZR = 1000
NZT = N // ZR
ZB = 50


def _norm(x):
    return jnp.sqrt(jnp.clip(jnp.sum(x * x, axis=-1, keepdims=True), 1e-15, None))


def _artanh(z):
    z = jnp.clip(z, -_MAX, _MAX)
    return 0.5 * jnp.log((1.0 + z) / (1.0 - z))


def _mobius_add(x, y):
    x2 = jnp.sum(x * x, -1, keepdims=True)
    y2 = jnp.sum(y * y, -1, keepdims=True)
    xy = jnp.sum(x * y, -1, keepdims=True)
    num = (1.0 + 2.0 * xy + y2) * x + (1.0 - x2) * y
    den = 1.0 + 2.0 * xy + x2 * y2
    return num / jnp.clip(den, 1e-15, None)


def _pdist(x, y):
    return 2.0 * _artanh(jnp.squeeze(_norm(_mobius_add(-x, y)), -1))


def _expmap0(u):
    n = _norm(u)
    return jnp.tanh(n) * u / n


def _logmap0(x):
    n = _norm(x)
    return _artanh(n) * x / n


def _pre_body(x_ref, xe_ref, wh_ref, we_ref, bh_ref, be_ref, asrc_ref,
              adst_ref, ht_ref, h_ref, he_ref, hn2_ref, sa_ref, sb_ref):
    x = x_ref[...]
    u = _logmap0(x)
    ht = jnp.dot(u, wh_ref[...], preferred_element_type=jnp.float32,
                 precision=lax.Precision.HIGHEST) + bh_ref[...]
    ht_ref[...] = ht
    nt = _norm(ht)
    th = jnp.tanh(nt)
    h_ref[...] = th * ht / nt
    hn2_ref[...] = th * th
    he = jnp.dot(xe_ref[...], we_ref[...], preferred_element_type=jnp.float32,
                 precision=lax.Precision.HIGHEST) + be_ref[...]
    he_ref[...] = he
    sa_ref[...] = jnp.dot(he, asrc_ref[...], preferred_element_type=jnp.float32,
                          precision=lax.Precision.HIGHEST)
    sb_ref[...] = jnp.dot(he, adst_ref[...], preferred_element_type=jnp.float32,
                          precision=lax.Precision.HIGHEST)


def _pre(x, x_e, W_h, b_h, W_e, b_e, a_src, a_dst):
    R = 1000
    grid = N // R
    row = lambda i: (i, 0)
    fixed = lambda i: (0, 0)
    return pl.pallas_call(
        _pre_body,
        grid=(grid,),
        in_specs=[
            pl.BlockSpec((R, D), row),
            pl.BlockSpec((R, D), row),
            pl.BlockSpec((D, D), fixed),
            pl.BlockSpec((D, D), fixed),
            pl.BlockSpec((1, D), fixed),
            pl.BlockSpec((1, D), fixed),
            pl.BlockSpec((D, 1), fixed),
            pl.BlockSpec((D, 1), fixed),
        ],
        out_specs=[
            pl.BlockSpec((R, D), row),
            pl.BlockSpec((R, D), row),
            pl.BlockSpec((R, D), row),
            pl.BlockSpec((R, 1), row),
            pl.BlockSpec((R, 1), row),
            pl.BlockSpec((R, 1), row),
        ],
        out_shape=[
            jax.ShapeDtypeStruct((N, D), jnp.float32),
            jax.ShapeDtypeStruct((N, D), jnp.float32),
            jax.ShapeDtypeStruct((N, D), jnp.float32),
            jax.ShapeDtypeStruct((N, 1), jnp.float32),
            jax.ShapeDtypeStruct((N, 1), jnp.float32),
            jax.ShapeDtypeStruct((N, 1), jnp.float32),
        ],
    )(x, x_e, W_h, W_e, b_h.reshape(1, D), b_e.reshape(1, D),
      a_src.reshape(D, 1), a_dst.reshape(D, 1))


def _sca_body(hp_hbm, src_hbm, dst_hbm, hn2_hbm, nm2_hbm,
              hn2_v, sidx_v, didx_v, nm2b_v, rows0_v, rows1_v,
              cidx0_v, cidx1_v, xyt_v, sem0, sem1):
    cid = lax.axis_index("c")
    sid = lax.axis_index("s")
    wid = sid * NC + cid
    base = wid * EW
    pltpu.sync_copy(hn2_hbm, hn2_v)
    pltpu.sync_copy(src_hbm.at[pl.ds(base, EW)], sidx_v)
    pltpu.sync_copy(dst_hbm.at[pl.ds(base, EW)], didx_v)

    lanes = lax.iota(jnp.int32, 16)
    bufs = ((rows0_v, cidx0_v, sem0), (rows1_v, cidx1_v, sem1))

    def issue(k, p):
        rows_v, cidx_v, sem = bufs[p]
        off = k * BA
        for g in range(BA // 16):
            s16 = sidx_v[pl.ds(off + g * 16, 16)]
            d16 = didx_v[pl.ds(off + g * 16, 16)]
            pos = 2 * (g * 16 + lanes)
            plsc.store_scatter(cidx_v, [pos], s16)
            plsc.store_scatter(cidx_v, [pos + 1], d16)
        pltpu.async_copy(hp_hbm.at[cidx_v], rows_v, sem)

    def process(k, p):
        rows_v, cidx_v, sem = bufs[p]
        pltpu.make_async_copy(hp_hbm.at[cidx_v], rows_v, sem).wait()
        off = k * BA
        for g in range(BA // 16):
            def epart(i, c):
                e = g * 16 + i
                acc = jnp.zeros((16,), jnp.float32)
                for j in range(DP // 16):
                    ps = plsc.bitcast(rows_v[2 * e, pl.ds(j * 16, 16)],
                                      jnp.bfloat16)
                    pd = plsc.bitcast(rows_v[2 * e + 1, pl.ds(j * 16, 16)],
                                      jnp.bfloat16)
                    sa_, sb_ = plsc.unpack(ps, format=plsc.PackFormat.INTERLEAVED)
                    da_, db_ = plsc.unpack(pd, format=plsc.PackFormat.INTERLEAVED)
                    acc = acc + sa_ * da_ + sb_ * db_
                plsc.store_scatter(xyt_v, [lanes * 16 + i], acc)
                return c

            lax.fori_loop(0, 16, epart, 0)
            xy = xyt_v[pl.ds(0, 16)]
            for l in range(1, 16):
                xy = xy + xyt_v[pl.ds(l * 16, 16)]
            s16 = sidx_v[pl.ds(off + g * 16, 16)]
            d16 = didx_v[pl.ds(off + g * 16, 16)]
            x2 = plsc.load_gather(hn2_v, [s16])
            y2 = plsc.load_gather(hn2_v, [d16])
            a = 1.0 - 2.0 * xy + y2
            b = 1.0 - x2
            den = 1.0 - 2.0 * xy + x2 * y2
            den = jnp.maximum(den, 1e-15)
            nm2b_v[pl.ds(off + g * 16, 16)] = (
                (a * a * x2 - 2.0 * a * b * xy + b * b * y2) / (den * den))

    issue(0, 0)

    def pair(t, carry):
        k0 = 2 * t
        issue(k0 + 1, 1)
        process(k0, 0)

        @pl.when(t < NCA // 2 - 1)
        def _():
            issue(k0 + 2, 0)

        process(k0 + 1, 1)
        return carry

    lax.fori_loop(0, NCA // 2, pair, 0)
    process_tail = NCA % 2
    if process_tail:
        issue(NCA - 1, 0)
        process(NCA - 1, 0)
    pltpu.sync_copy(nm2b_v, nm2_hbm.at[pl.ds(base, EW)])


def _sca(hp, src, dst, hn2):
    mesh = plsc.VectorSubcoreMesh(core_axis_name="c", subcore_axis_name="s")
    f = pl.kernel(
        _sca_body,
        out_type=jax.ShapeDtypeStruct((E,), jnp.float32),
        mesh=mesh,
        compiler_params=pltpu.CompilerParams(needs_layout_passes=False,
                                             use_tc_tiling_on_sc=False),
        scratch_types=[
            pltpu.VMEM((N,), jnp.float32),
            pltpu.VMEM((EW,), jnp.int32),
            pltpu.VMEM((EW,), jnp.int32),
            pltpu.VMEM((EW,), jnp.float32),
            pltpu.VMEM((2 * BA, DP), jnp.int32),
            pltpu.VMEM((2 * BA, DP), jnp.int32),
            pltpu.VMEM((2 * BA,), jnp.int32),
            pltpu.VMEM((2 * BA,), jnp.int32),
            pltpu.VMEM((256,), jnp.float32),
            pltpu.SemaphoreType.DMA,
            pltpu.SemaphoreType.DMA,
        ],
    )
    return f(hp, src, dst, hn2)


def _tcb_body(nm2_ref, ah_ref, w_ref):
    n = jnp.sqrt(jnp.clip(nm2_ref[...], 1e-15, None))
    pd = 2.0 * _artanh(n)
    z = -ah_ref[0, 0] * pd
    z = jnp.where(z >= 0.0, z, 0.2 * z)
    w_ref[...] = jnp.exp(z)


def _tcb(nm2, a_h):
    RE = E // D
    return pl.pallas_call(
        _tcb_body,
        in_specs=[
            pl.BlockSpec((RE, D), lambda: (0, 0)),
            pl.BlockSpec(memory_space=pltpu.SMEM),
        ],
        out_specs=pl.BlockSpec((RE, D), lambda: (0, 0)),
        out_shape=jax.ShapeDtypeStruct((RE, D), jnp.float32),
    )(nm2.reshape(RE, D), a_h.reshape(1, 1))


def _scc_body(ht_hbm, he_hbm, wh_hbm, src_hbm, dst_hbm, sa_hbm, sb_hbm,
              acch_hbm, dh_hbm, acce_hbm, de_hbm,
              acc_sh, accd_sh, auxa_v, auxb_v,
              sidx_vs, didx_vs, w_vs, rows_vs, isems, gsems):
    cid = lax.axis_index("c")
    sid = lax.axis_index("s")
    ebase = sid * ET

    def zzero(r, c):
        for j in range(D // 16):
            rows_vs[0][r, pl.ds(j * 16, 16)] = jnp.zeros((16,), jnp.float32)
        return c

    lax.fori_loop(0, BC, zzero, 0)
    for g in range(BC // 16):
        w_vs[0][pl.ds(g * 16, 16)] = jnp.zeros((16,), jnp.float32)

    @pl.when(sid < NZT)
    def _():
        r0 = sid * ZR
        for q in range(ZR // BC):
            pltpu.sync_copy(rows_vs[0], acc_sh.at[pl.ds(r0 + q * BC, BC)])
            pltpu.sync_copy(w_vs[0], accd_sh.at[pl.ds(r0 + q * BC, BC)])
        rem = ZR % BC
        if rem:
            q0 = r0 + (ZR // BC) * BC
            pltpu.sync_copy(rows_vs[0].at[pl.ds(0, rem)],
                            acc_sh.at[pl.ds(q0, rem)])
            pltpu.sync_copy(w_vs[0].at[pl.ds(0, rem)],
                            accd_sh.at[pl.ds(q0, rem)])

    @pl.when(cid == 1)
    def _():
        pltpu.sync_copy(sa_hbm, auxa_v)
        pltpu.sync_copy(sb_hbm, auxb_v)

    plsc.subcore_barrier()

    def mk_idx_copies(c, q, compute_w):
        off = ebase + c * BC
        cps = [
            pltpu.make_async_copy(src_hbm.at[pl.ds(off, BC)], sidx_vs[q],
                                  isems[q]),
            pltpu.make_async_copy(dst_hbm.at[pl.ds(off, BC)], didx_vs[q],
                                  isems[q]),
        ]
        if not compute_w:
            cps.append(pltpu.make_async_copy(wh_hbm.at[pl.ds(off, BC)],
                                             w_vs[q], isems[q]))
        return cps

    def issue_idx(c, q, compute_w):
        for cp in mk_idx_copies(c, q, compute_w):
            cp.start()

    def wait_idx(c, q, compute_w):
        for cp in mk_idx_copies(c, q, compute_w):
            cp.wait()

    def issue_gather(q, p, rows_hbm):
        pltpu.async_copy(rows_hbm.at[sidx_vs[q]], rows_vs[p], gsems[p])

    def wait_scatter(q, p):
        pltpu.make_async_copy(rows_vs[p], acc_sh.at[didx_vs[q]],
                              gsems[p]).wait()
        pltpu.make_async_copy(w_vs[q], accd_sh.at[didx_vs[q]],
                              gsems[p]).wait()

    def process(q, p, rows_hbm, compute_w):
        rows_v = rows_vs[p]
        w_s = w_vs[q] if compute_w else w_vs[q]
        if compute_w:
            for g in range(BC // 16):
                s16 = sidx_vs[q][pl.ds(g * 16, 16)]
                d16 = didx_vs[q][pl.ds(g * 16, 16)]
                lg = (plsc.load_gather(auxa_v, [s16])
                      + plsc.load_gather(auxb_v, [d16]))
                lg = jnp.where(lg >= 0.0, lg, 0.2 * lg)
                w_s[pl.ds(g * 16, 16)] = jnp.exp(lg)
        pltpu.make_async_copy(rows_hbm.at[sidx_vs[q]], rows_v,
                              gsems[p]).wait()

        def scale(t, c):
            for u in range(2):
                e = 2 * t + u
                wv = plsc.load_gather(w_s, [jnp.zeros((16,), jnp.int32) + e])
                for j in range(D // 16):
                    rows_v[e, pl.ds(j * 16, 16)] = (
                        rows_v[e, pl.ds(j * 16, 16)] * wv)
            return c

        lax.fori_loop(0, BC // 2, scale, 0)
        pltpu.async_copy(rows_v, acc_sh.at[didx_vs[q]], gsems[p], add=True)
        pltpu.async_copy(w_s, accd_sh.at[didx_vs[q]], gsems[p], add=True)

    def run(rows_hbm, compute_w):
        issue_idx(0, 0, compute_w)
        issue_idx(1, 1, compute_w)
        wait_idx(0, 0, compute_w)
        issue_gather(0, 0, rows_hbm)

        def step(c, i):
            q, p = i % 4, i % 2
            qn, pn = (i + 1) % 4, (i + 1) % 2

            @pl.when(c >= 1)
            def _():
                wait_scatter((i - 1) % 4, (i - 1) % 2)

            @pl.when(c + 1 < NCC)
            def _():
                wait_idx(c + 1, qn, compute_w)
                issue_gather(qn, pn, rows_hbm)

            @pl.when(c + 2 < NCC)
            def _():
                issue_idx(c + 2, (i + 2) % 4, compute_w)

            process(q, p, rows_hbm, compute_w)

        def quad(t, carry):
            c0 = 4 * t
            for i in range(4):
                step(c0 + i, i)
            return carry

        lax.fori_loop(0, NCC // 4, quad, 0)
        for i in range(NCC % 4):
            step((NCC // 4) * 4 + i, i)
        wait_scatter((NCC - 1) % 4, (NCC - 1) % 2)

    @pl.when(cid == 0)
    def _():
        run(ht_hbm, False)

    @pl.when(cid == 1)
    def _():
        run(he_hbm, True)

    plsc.subcore_barrier()

    @pl.when(sid < NZT)
    def _():
        r0 = sid * ZR

        def wb(vals_hbm, d_hbm):
            pltpu.sync_copy(acc_sh.at[pl.ds(r0, ZR)], vals_hbm.at[pl.ds(r0, ZR)])
            for q in range(ZR // BC):
                pltpu.sync_copy(accd_sh.at[pl.ds(r0 + q * BC, BC)], w_vs[0])
                pltpu.sync_copy(w_vs[0], d_hbm.at[pl.ds(r0 + q * BC, BC)])
            rem = ZR % BC
            if rem:
                q0 = r0 + (ZR // BC) * BC
                pltpu.sync_copy(accd_sh.at[pl.ds(q0, rem)],
                                w_vs[0].at[pl.ds(0, rem)])
                pltpu.sync_copy(w_vs[0].at[pl.ds(0, rem)],
                                d_hbm.at[pl.ds(q0, rem)])

        @pl.when(cid == 0)
        def _():
            wb(acch_hbm, dh_hbm)

        @pl.when(cid == 1)
        def _():
            wb(acce_hbm, de_hbm)


def _scc(ht, he, wh, src, dst, sa, sb):
    mesh = plsc.VectorSubcoreMesh(core_axis_name="c", subcore_axis_name="s")
    f = pl.kernel(
        _scc_body,
        out_type=[
            jax.ShapeDtypeStruct((N, D), jnp.float32),
            jax.ShapeDtypeStruct((N,), jnp.float32),
            jax.ShapeDtypeStruct((N, D), jnp.float32),
            jax.ShapeDtypeStruct((N,), jnp.float32),
        ],
        mesh=mesh,
        compiler_params=pltpu.CompilerParams(needs_layout_passes=False),
        scratch_types=[
            pltpu.VMEM_SHARED((N, D), jnp.float32),
            pltpu.VMEM_SHARED((N,), jnp.float32),
            pltpu.VMEM((N,), jnp.float32),
            pltpu.VMEM((N,), jnp.float32),
            [pltpu.VMEM((BC,), jnp.int32) for _ in range(4)],
            [pltpu.VMEM((BC,), jnp.int32) for _ in range(4)],
            [pltpu.VMEM((BC,), jnp.float32) for _ in range(4)],
            [pltpu.VMEM((BC, D), jnp.float32) for _ in range(2)],
            [pltpu.SemaphoreType.DMA for _ in range(4)],
            [pltpu.SemaphoreType.DMA for _ in range(2)],
        ],
    )
    return f(ht, he, wh, src, dst, sa, sb)


def _tcd_body(acch_ref, dh_ref, acce_ref, de_ref, athf_ref, atef_ref,
              xh_ref, xe_ref):
    agg_h = acch_ref[...] / (dh_ref[...] + 1e-16)
    x_h = _expmap0(jnp.maximum(agg_h, 0.0))
    xe2 = jnp.maximum(acce_ref[...] / (de_ref[...] + 1e-16), 0.0)
    xe_hyp = _expmap0(xe2)
    dist_f = _pdist(x_h, xe_hyp) * athf_ref[0, 0]
    nx = _norm(xe_hyp)
    ms = jnp.tanh(dist_f[:, None] * _artanh(nx)) * xe_hyp / nx
    x_h = _mobius_add(x_h, ms)
    xh_ref[...] = x_h
    log_xh = _logmap0(x_h)
    dist_e = jnp.sum((log_xh - xe2) ** 2, axis=-1, keepdims=True) * atef_ref[0, 0]
    xe_ref[...] = xe2 + dist_e * log_xh


def _tcd(acch, dh, acce, de, att_hf, att_ef):
    R = 1000
    grid = N // R
    row = lambda i: (i, 0)
    return pl.pallas_call(
        _tcd_body,
        grid=(grid,),
        in_specs=[
            pl.BlockSpec((R, D), row),
            pl.BlockSpec((R, 1), row),
            pl.BlockSpec((R, D), row),
            pl.BlockSpec((R, 1), row),
            pl.BlockSpec(memory_space=pltpu.SMEM),
            pl.BlockSpec(memory_space=pltpu.SMEM),
        ],
        out_specs=[
            pl.BlockSpec((R, D), row),
            pl.BlockSpec((R, D), row),
        ],
        out_shape=[
            jax.ShapeDtypeStruct((N, D), jnp.float32),
            jax.ShapeDtypeStruct((N, D), jnp.float32),
        ],
    )(acch, dh, acce, de, att_hf.reshape(1, 1), att_ef.reshape(1, 1))


def kernel(x, x_e, edge_index, W_h, b_h, a_h, W_e, b_e, a_src, a_dst,
           att_hf, att_ef):
    src = edge_index[0]
    dst = edge_index[1]
    ht, h, he, hn2, sa, sb = _pre(x, x_e, W_h, b_h, W_e, b_e, a_src, a_dst)
    hp = lax.bitcast_convert_type(
        h.astype(jnp.bfloat16).reshape(N, DP, 2), jnp.int32)
    nm2 = _sca(hp, src, dst, hn2.reshape(N))
    wh = _tcb(nm2, a_h).reshape(E)
    acch, dh, acce, de = _scc(ht, he, wh, src, dst,
                              sa.reshape(N), sb.reshape(N))
    return _tcd(acch, dh.reshape(N, 1), acce, de.reshape(N, 1),
                att_hf, att_ef)


# speedup vs baseline: 1.0347x; 1.0347x over previous
import functools

import jax
import jax.numpy as jnp
from jax import lax
from jax.experimental import pallas as pl
from jax.experimental.pallas import tpu as pltpu
from jax.experimental.pallas import tpu_sc as plsc

N = 10000
E = 320000
D = 128
_MAX = 1.0 - 1e-5

NC = 2
NS = 16
NW = NC * NS
EW = E // NW
BA = 80
DP = D // 2
NCA = EW // BA
ET = E // NS
BC = 80
NCC = ET // BC
ZR = 1000
NZT = N // ZR
ZB = 50


def _norm(x):
    return jnp.sqrt(jnp.clip(jnp.sum(x * x, axis=-1, keepdims=True), 1e-15, None))


def _artanh(z):
    z = jnp.clip(z, -_MAX, _MAX)
    return 0.5 * jnp.log((1.0 + z) / (1.0 - z))


def _mobius_add(x, y):
    x2 = jnp.sum(x * x, -1, keepdims=True)
    y2 = jnp.sum(y * y, -1, keepdims=True)
    xy = jnp.sum(x * y, -1, keepdims=True)
    num = (1.0 + 2.0 * xy + y2) * x + (1.0 - x2) * y
    den = 1.0 + 2.0 * xy + x2 * y2
    return num / jnp.clip(den, 1e-15, None)


def _pdist(x, y):
    return 2.0 * _artanh(jnp.squeeze(_norm(_mobius_add(-x, y)), -1))


def _expmap0(u):
    n = _norm(u)
    return jnp.tanh(n) * u / n


def _logmap0(x):
    n = _norm(x)
    return _artanh(n) * x / n


def _pre_body(x_ref, xe_ref, wh_ref, we_ref, bh_ref, be_ref, asrc_ref,
              adst_ref, ht_ref, h_ref, he_ref, hn2_ref, sa_ref, sb_ref):
    x = x_ref[...]
    u = _logmap0(x)
    ht = jnp.dot(u, wh_ref[...], preferred_element_type=jnp.float32,
                 precision=lax.Precision.HIGHEST) + bh_ref[...]
    ht_ref[...] = ht
    nt = _norm(ht)
    th = jnp.tanh(nt)
    h_ref[...] = th * ht / nt
    hn2_ref[...] = th * th
    he = jnp.dot(xe_ref[...], we_ref[...], preferred_element_type=jnp.float32,
                 precision=lax.Precision.HIGHEST) + be_ref[...]
    he_ref[...] = he
    sa_ref[...] = jnp.dot(he, asrc_ref[...], preferred_element_type=jnp.float32,
                          precision=lax.Precision.HIGHEST)
    sb_ref[...] = jnp.dot(he, adst_ref[...], preferred_element_type=jnp.float32,
                          precision=lax.Precision.HIGHEST)


def _pre(x, x_e, W_h, b_h, W_e, b_e, a_src, a_dst):
    R = 1000
    grid = N // R
    row = lambda i: (i, 0)
    fixed = lambda i: (0, 0)
    return pl.pallas_call(
        _pre_body,
        grid=(grid,),
        in_specs=[
            pl.BlockSpec((R, D), row),
            pl.BlockSpec((R, D), row),
            pl.BlockSpec((D, D), fixed),
            pl.BlockSpec((D, D), fixed),
            pl.BlockSpec((1, D), fixed),
            pl.BlockSpec((1, D), fixed),
            pl.BlockSpec((D, 1), fixed),
            pl.BlockSpec((D, 1), fixed),
        ],
        out_specs=[
            pl.BlockSpec((R, D), row),
            pl.BlockSpec((R, D), row),
            pl.BlockSpec((R, D), row),
            pl.BlockSpec((R, 1), row),
            pl.BlockSpec((R, 1), row),
            pl.BlockSpec((R, 1), row),
        ],
        out_shape=[
            jax.ShapeDtypeStruct((N, D), jnp.float32),
            jax.ShapeDtypeStruct((N, D), jnp.float32),
            jax.ShapeDtypeStruct((N, D), jnp.float32),
            jax.ShapeDtypeStruct((N, 1), jnp.float32),
            jax.ShapeDtypeStruct((N, 1), jnp.float32),
            jax.ShapeDtypeStruct((N, 1), jnp.float32),
        ],
    )(x, x_e, W_h, W_e, b_h.reshape(1, D), b_e.reshape(1, D),
      a_src.reshape(D, 1), a_dst.reshape(D, 1))


def _sca_body(h_hbm, src_hbm, dst_hbm, hn2_hbm, nm2_hbm,
              hn2_v, sidx_v, didx_v, nm2b_v, rows0_v, rows1_v,
              cidx0_v, cidx1_v, xyt_v, sem0, sem1):
    cid = lax.axis_index("c")
    sid = lax.axis_index("s")
    wid = sid * NC + cid
    base = wid * EW
    pltpu.sync_copy(hn2_hbm, hn2_v)
    pltpu.sync_copy(src_hbm.at[pl.ds(base, EW)], sidx_v)
    pltpu.sync_copy(dst_hbm.at[pl.ds(base, EW)], didx_v)

    lanes = lax.iota(jnp.int32, 16)
    bufs = ((rows0_v, cidx0_v, sem0), (rows1_v, cidx1_v, sem1))

    def issue(k, p):
        rows_v, cidx_v, sem = bufs[p]
        off = k * BA
        for g in range(BA // 16):
            s16 = sidx_v[pl.ds(off + g * 16, 16)]
            d16 = didx_v[pl.ds(off + g * 16, 16)]
            pos = 2 * (g * 16 + lanes)
            plsc.store_scatter(cidx_v, [pos], s16)
            plsc.store_scatter(cidx_v, [pos + 1], d16)
        pltpu.async_copy(h_hbm.at[cidx_v], rows_v, sem)

    def process(k, p):
        rows_v, cidx_v, sem = bufs[p]
        pltpu.make_async_copy(h_hbm.at[cidx_v], rows_v, sem).wait()
        off = k * BA
        for g in range(BA // 16):
            def epart(i, c):
                e = g * 16 + i
                acc = rows_v[2 * e, pl.ds(0, 16)] * rows_v[2 * e + 1, pl.ds(0, 16)]
                for j in range(1, D // 16):
                    acc = acc + (rows_v[2 * e, pl.ds(j * 16, 16)]
                                 * rows_v[2 * e + 1, pl.ds(j * 16, 16)])
                plsc.store_scatter(xyt_v, [lanes * 16 + i], acc)
                return c

            lax.fori_loop(0, 16, epart, 0)
            xy = xyt_v[pl.ds(0, 16)]
            for l in range(1, 16):
                xy = xy + xyt_v[pl.ds(l * 16, 16)]
            s16 = sidx_v[pl.ds(off + g * 16, 16)]
            d16 = didx_v[pl.ds(off + g * 16, 16)]
            x2 = plsc.load_gather(hn2_v, [s16])
            y2 = plsc.load_gather(hn2_v, [d16])
            a = 1.0 - 2.0 * xy + y2
            b = 1.0 - x2
            den = 1.0 - 2.0 * xy + x2 * y2
            den = jnp.maximum(den, 1e-15)
            nm2b_v[pl.ds(off + g * 16, 16)] = (
                (a * a * x2 - 2.0 * a * b * xy + b * b * y2) / (den * den))

    issue(0, 0)

    def pair(t, carry):
        k0 = 2 * t
        issue(k0 + 1, 1)
        process(k0, 0)

        @pl.when(t < NCA // 2 - 1)
        def _():
            issue(k0 + 2, 0)

        process(k0 + 1, 1)
        return carry

    lax.fori_loop(0, NCA // 2, pair, 0)
    process_tail = NCA % 2
    if process_tail:
        issue(NCA - 1, 0)
        process(NCA - 1, 0)
    pltpu.sync_copy(nm2b_v, nm2_hbm.at[pl.ds(base, EW)])


def _sca(h, src, dst, hn2):
    mesh = plsc.VectorSubcoreMesh(core_axis_name="c", subcore_axis_name="s")
    f = pl.kernel(
        _sca_body,
        out_type=jax.ShapeDtypeStruct((E,), jnp.float32),
        mesh=mesh,
        compiler_params=pltpu.CompilerParams(needs_layout_passes=False),
        scratch_types=[
            pltpu.VMEM((N,), jnp.float32),
            pltpu.VMEM((EW,), jnp.int32),
            pltpu.VMEM((EW,), jnp.int32),
            pltpu.VMEM((EW,), jnp.float32),
            pltpu.VMEM((2 * BA, D), jnp.float32),
            pltpu.VMEM((2 * BA, D), jnp.float32),
            pltpu.VMEM((2 * BA,), jnp.int32),
            pltpu.VMEM((2 * BA,), jnp.int32),
            pltpu.VMEM((256,), jnp.float32),
            pltpu.SemaphoreType.DMA,
            pltpu.SemaphoreType.DMA,
        ],
    )
    return f(h, src, dst, hn2)


def _tcb_body(nm2_ref, ah_ref, w_ref):
    n = jnp.sqrt(jnp.clip(nm2_ref[...], 1e-15, None))
    pd = 2.0 * _artanh(n)
    z = -ah_ref[0, 0] * pd
    z = jnp.where(z >= 0.0, z, 0.2 * z)
    w_ref[...] = jnp.exp(z)


def _tcb(nm2, a_h):
    RE = E // D
    return pl.pallas_call(
        _tcb_body,
        in_specs=[
            pl.BlockSpec((RE, D), lambda: (0, 0)),
            pl.BlockSpec(memory_space=pltpu.SMEM),
        ],
        out_specs=pl.BlockSpec((RE, D), lambda: (0, 0)),
        out_shape=jax.ShapeDtypeStruct((RE, D), jnp.float32),
    )(nm2.reshape(RE, D), a_h.reshape(1, 1))


def _scc_body(ht_hbm, he_hbm, wh_hbm, src_hbm, dst_hbm, sa_hbm, sb_hbm,
              acch_hbm, dh_hbm, acce_hbm, de_hbm,
              acc_sh, accd_sh, auxa_v, auxb_v,
              sidx_vs, didx_vs, w_vs, rows_vs, isems, gsems):
    cid = lax.axis_index("c")
    sid = lax.axis_index("s")
    ebase = sid * ET

    def zzero(r, c):
        for j in range(D // 16):
            rows_vs[0][r, pl.ds(j * 16, 16)] = jnp.zeros((16,), jnp.float32)
        return c

    lax.fori_loop(0, BC, zzero, 0)
    for g in range(BC // 16):
        w_vs[0][pl.ds(g * 16, 16)] = jnp.zeros((16,), jnp.float32)

    @pl.when(sid < NZT)
    def _():
        r0 = sid * ZR
        for q in range(ZR // BC):
            pltpu.sync_copy(rows_vs[0], acc_sh.at[pl.ds(r0 + q * BC, BC)])
            pltpu.sync_copy(w_vs[0], accd_sh.at[pl.ds(r0 + q * BC, BC)])
        rem = ZR % BC
        if rem:
            q0 = r0 + (ZR // BC) * BC
            pltpu.sync_copy(rows_vs[0].at[pl.ds(0, rem)],
                            acc_sh.at[pl.ds(q0, rem)])
            pltpu.sync_copy(w_vs[0].at[pl.ds(0, rem)],
                            accd_sh.at[pl.ds(q0, rem)])

    @pl.when(cid == 1)
    def _():
        pltpu.sync_copy(sa_hbm, auxa_v)
        pltpu.sync_copy(sb_hbm, auxb_v)

    plsc.subcore_barrier()

    def mk_idx_copies(c, q, compute_w):
        off = ebase + c * BC
        cps = [
            pltpu.make_async_copy(src_hbm.at[pl.ds(off, BC)], sidx_vs[q],
                                  isems[q]),
            pltpu.make_async_copy(dst_hbm.at[pl.ds(off, BC)], didx_vs[q],
                                  isems[q]),
        ]
        if not compute_w:
            cps.append(pltpu.make_async_copy(wh_hbm.at[pl.ds(off, BC)],
                                             w_vs[q], isems[q]))
        return cps

    def issue_idx(c, q, compute_w):
        for cp in mk_idx_copies(c, q, compute_w):
            cp.start()

    def wait_idx(c, q, compute_w):
        for cp in mk_idx_copies(c, q, compute_w):
            cp.wait()

    def issue_gather(q, p, rows_hbm):
        pltpu.async_copy(rows_hbm.at[sidx_vs[q]], rows_vs[p], gsems[p])

    def wait_scatter(q, p):
        pltpu.make_async_copy(rows_vs[p], acc_sh.at[didx_vs[q]],
                              gsems[p]).wait()
        pltpu.make_async_copy(w_vs[q], accd_sh.at[didx_vs[q]],
                              gsems[p]).wait()

    def process(q, p, rows_hbm, compute_w):
        rows_v = rows_vs[p]
        w_s = w_vs[q] if compute_w else w_vs[q]
        if compute_w:
            for g in range(BC // 16):
                s16 = sidx_vs[q][pl.ds(g * 16, 16)]
                d16 = didx_vs[q][pl.ds(g * 16, 16)]
                lg = (plsc.load_gather(auxa_v, [s16])
                      + plsc.load_gather(auxb_v, [d16]))
                lg = jnp.where(lg >= 0.0, lg, 0.2 * lg)
                w_s[pl.ds(g * 16, 16)] = jnp.exp(lg)
        pltpu.make_async_copy(rows_hbm.at[sidx_vs[q]], rows_v,
                              gsems[p]).wait()

        def scale(t, c):
            for u in range(4):
                e = 4 * t + u
                wv = plsc.load_gather(w_s, [jnp.zeros((16,), jnp.int32) + e])
                for j in range(D // 16):
                    rows_v[e, pl.ds(j * 16, 16)] = (
                        rows_v[e, pl.ds(j * 16, 16)] * wv)
            return c

        lax.fori_loop(0, BC // 4, scale, 0)
        pltpu.async_copy(rows_v, acc_sh.at[didx_vs[q]], gsems[p], add=True)
        pltpu.async_copy(w_s, accd_sh.at[didx_vs[q]], gsems[p], add=True)

    def run(rows_hbm, compute_w):
        issue_idx(0, 0, compute_w)
        issue_idx(1, 1, compute_w)
        wait_idx(0, 0, compute_w)
        issue_gather(0, 0, rows_hbm)

        def step(c, i):
            q, p = i % 4, i % 2
            qn, pn = (i + 1) % 4, (i + 1) % 2

            @pl.when(c >= 1)
            def _():
                wait_scatter((i - 1) % 4, (i - 1) % 2)

            @pl.when(c + 1 < NCC)
            def _():
                wait_idx(c + 1, qn, compute_w)
                issue_gather(qn, pn, rows_hbm)

            @pl.when(c + 2 < NCC)
            def _():
                issue_idx(c + 2, (i + 2) % 4, compute_w)

            process(q, p, rows_hbm, compute_w)

        def quad(t, carry):
            c0 = 4 * t
            for i in range(4):
                step(c0 + i, i)
            return carry

        lax.fori_loop(0, NCC // 4, quad, 0)
        for i in range(NCC % 4):
            step((NCC // 4) * 4 + i, i)
        wait_scatter((NCC - 1) % 4, (NCC - 1) % 2)

    @pl.when(cid == 0)
    def _():
        run(ht_hbm, False)

    @pl.when(cid == 1)
    def _():
        run(he_hbm, True)

    plsc.subcore_barrier()

    @pl.when(sid < NZT)
    def _():
        r0 = sid * ZR

        def wb(vals_hbm, d_hbm):
            pltpu.sync_copy(acc_sh.at[pl.ds(r0, ZR)], vals_hbm.at[pl.ds(r0, ZR)])
            for q in range(ZR // BC):
                pltpu.sync_copy(accd_sh.at[pl.ds(r0 + q * BC, BC)], w_vs[0])
                pltpu.sync_copy(w_vs[0], d_hbm.at[pl.ds(r0 + q * BC, BC)])
            rem = ZR % BC
            if rem:
                q0 = r0 + (ZR // BC) * BC
                pltpu.sync_copy(accd_sh.at[pl.ds(q0, rem)],
                                w_vs[0].at[pl.ds(0, rem)])
                pltpu.sync_copy(w_vs[0].at[pl.ds(0, rem)],
                                d_hbm.at[pl.ds(q0, rem)])

        @pl.when(cid == 0)
        def _():
            wb(acch_hbm, dh_hbm)

        @pl.when(cid == 1)
        def _():
            wb(acce_hbm, de_hbm)


def _scc(ht, he, wh, src, dst, sa, sb):
    mesh = plsc.VectorSubcoreMesh(core_axis_name="c", subcore_axis_name="s")
    f = pl.kernel(
        _scc_body,
        out_type=[
            jax.ShapeDtypeStruct((N, D), jnp.float32),
            jax.ShapeDtypeStruct((N,), jnp.float32),
            jax.ShapeDtypeStruct((N, D), jnp.float32),
            jax.ShapeDtypeStruct((N,), jnp.float32),
        ],
        mesh=mesh,
        compiler_params=pltpu.CompilerParams(needs_layout_passes=False),
        scratch_types=[
            pltpu.VMEM_SHARED((N, D), jnp.float32),
            pltpu.VMEM_SHARED((N,), jnp.float32),
            pltpu.VMEM((N,), jnp.float32),
            pltpu.VMEM((N,), jnp.float32),
            [pltpu.VMEM((BC,), jnp.int32) for _ in range(4)],
            [pltpu.VMEM((BC,), jnp.int32) for _ in range(4)],
            [pltpu.VMEM((BC,), jnp.float32) for _ in range(4)],
            [pltpu.VMEM((BC, D), jnp.float32) for _ in range(2)],
            [pltpu.SemaphoreType.DMA for _ in range(4)],
            [pltpu.SemaphoreType.DMA for _ in range(2)],
        ],
    )
    return f(ht, he, wh, src, dst, sa, sb)


def _tcd_body(acch_ref, dh_ref, acce_ref, de_ref, athf_ref, atef_ref,
              xh_ref, xe_ref):
    agg_h = acch_ref[...] / (dh_ref[...] + 1e-16)
    x_h = _expmap0(jnp.maximum(agg_h, 0.0))
    xe2 = jnp.maximum(acce_ref[...] / (de_ref[...] + 1e-16), 0.0)
    xe_hyp = _expmap0(xe2)
    dist_f = _pdist(x_h, xe_hyp) * athf_ref[0, 0]
    nx = _norm(xe_hyp)
    ms = jnp.tanh(dist_f[:, None] * _artanh(nx)) * xe_hyp / nx
    x_h = _mobius_add(x_h, ms)
    xh_ref[...] = x_h
    log_xh = _logmap0(x_h)
    dist_e = jnp.sum((log_xh - xe2) ** 2, axis=-1, keepdims=True) * atef_ref[0, 0]
    xe_ref[...] = xe2 + dist_e * log_xh


def _tcd(acch, dh, acce, de, att_hf, att_ef):
    R = 1000
    grid = N // R
    row = lambda i: (i, 0)
    return pl.pallas_call(
        _tcd_body,
        grid=(grid,),
        in_specs=[
            pl.BlockSpec((R, D), row),
            pl.BlockSpec((R, 1), row),
            pl.BlockSpec((R, D), row),
            pl.BlockSpec((R, 1), row),
            pl.BlockSpec(memory_space=pltpu.SMEM),
            pl.BlockSpec(memory_space=pltpu.SMEM),
        ],
        out_specs=[
            pl.BlockSpec((R, D), row),
            pl.BlockSpec((R, D), row),
        ],
        out_shape=[
            jax.ShapeDtypeStruct((N, D), jnp.float32),
            jax.ShapeDtypeStruct((N, D), jnp.float32),
        ],
    )(acch, dh, acce, de, att_hf.reshape(1, 1), att_ef.reshape(1, 1))


def kernel(x, x_e, edge_index, W_h, b_h, a_h, W_e, b_e, a_src, a_dst,
           att_hf, att_ef):
    src = edge_index[0]
    dst = edge_index[1]
    ht, h, he, hn2, sa, sb = _pre(x, x_e, W_h, b_h, W_e, b_e, a_src, a_dst)
    nm2 = _sca(h, src, dst, hn2.reshape(N))
    wh = _tcb(nm2, a_h).reshape(E)
    acch, dh, acce, de = _scc(ht, he, wh, src, dst,
                              sa.reshape(N), sb.reshape(N))
    return _tcd(acch, dh.reshape(N, 1), acce, de.reshape(N, 1),
                att_hf, att_ef)


# speedup vs baseline: 1.0694x; 1.0335x over previous
import functools

import jax
import jax.numpy as jnp
from jax import lax
from jax.experimental import pallas as pl
from jax.experimental.pallas import tpu as pltpu
from jax.experimental.pallas import tpu_sc as plsc

N = 10000
E = 320000
D = 128
_MAX = 1.0 - 1e-5

NC = 2
NS = 16
NW = NC * NS
EW = E // NW
BA = 80
DP = D // 2
NCA = EW // BA
ET = E // NS
BC = 80
NCC = ET // BC
ZR = 1000
NZT = N // ZR
ZB = 50


def _norm(x):
    return jnp.sqrt(jnp.clip(jnp.sum(x * x, axis=-1, keepdims=True), 1e-15, None))


def _artanh(z):
    z = jnp.clip(z, -_MAX, _MAX)
    return 0.5 * jnp.log((1.0 + z) / (1.0 - z))


def _mobius_add(x, y):
    x2 = jnp.sum(x * x, -1, keepdims=True)
    y2 = jnp.sum(y * y, -1, keepdims=True)
    xy = jnp.sum(x * y, -1, keepdims=True)
    num = (1.0 + 2.0 * xy + y2) * x + (1.0 - x2) * y
    den = 1.0 + 2.0 * xy + x2 * y2
    return num / jnp.clip(den, 1e-15, None)


def _pdist(x, y):
    return 2.0 * _artanh(jnp.squeeze(_norm(_mobius_add(-x, y)), -1))


def _expmap0(u):
    n = _norm(u)
    return jnp.tanh(n) * u / n


def _logmap0(x):
    n = _norm(x)
    return _artanh(n) * x / n


def _pre_body(x_ref, xe_ref, wh_ref, we_ref, bh_ref, be_ref, asrc_ref,
              adst_ref, ht_ref, h_ref, he_ref, hn2_ref, sa_ref, sb_ref):
    x = x_ref[...]
    u = _logmap0(x)
    ht = jnp.dot(u, wh_ref[...], preferred_element_type=jnp.float32) + bh_ref[...]
    ht_ref[...] = ht
    nt = _norm(ht)
    th = jnp.tanh(nt)
    h_ref[...] = th * ht / nt
    hn2_ref[...] = th * th
    he = jnp.dot(xe_ref[...], we_ref[...], preferred_element_type=jnp.float32) + be_ref[...]
    he_ref[...] = he
    sa_ref[...] = jnp.dot(he, asrc_ref[...], preferred_element_type=jnp.float32)
    sb_ref[...] = jnp.dot(he, adst_ref[...], preferred_element_type=jnp.float32)


def _pre(x, x_e, W_h, b_h, W_e, b_e, a_src, a_dst):
    R = 1000
    grid = N // R
    row = lambda i: (i, 0)
    fixed = lambda i: (0, 0)
    return pl.pallas_call(
        _pre_body,
        grid=(grid,),
        in_specs=[
            pl.BlockSpec((R, D), row),
            pl.BlockSpec((R, D), row),
            pl.BlockSpec((D, D), fixed),
            pl.BlockSpec((D, D), fixed),
            pl.BlockSpec((1, D), fixed),
            pl.BlockSpec((1, D), fixed),
            pl.BlockSpec((D, 1), fixed),
            pl.BlockSpec((D, 1), fixed),
        ],
        out_specs=[
            pl.BlockSpec((R, D), row),
            pl.BlockSpec((R, D), row),
            pl.BlockSpec((R, D), row),
            pl.BlockSpec((R, 1), row),
            pl.BlockSpec((R, 1), row),
            pl.BlockSpec((R, 1), row),
        ],
        out_shape=[
            jax.ShapeDtypeStruct((N, D), jnp.float32),
            jax.ShapeDtypeStruct((N, D), jnp.float32),
            jax.ShapeDtypeStruct((N, D), jnp.float32),
            jax.ShapeDtypeStruct((N, 1), jnp.float32),
            jax.ShapeDtypeStruct((N, 1), jnp.float32),
            jax.ShapeDtypeStruct((N, 1), jnp.float32),
        ],
    )(x, x_e, W_h, W_e, b_h.reshape(1, D), b_e.reshape(1, D),
      a_src.reshape(D, 1), a_dst.reshape(D, 1))


def _sca_body(h_hbm, src_hbm, dst_hbm, hn2_hbm, nm2_hbm,
              hn2_v, sidx_v, didx_v, nm2b_v, rows0_v, rows1_v,
              cidx0_v, cidx1_v, xyt_v, sem0, sem1):
    cid = lax.axis_index("c")
    sid = lax.axis_index("s")
    wid = sid * NC + cid
    base = wid * EW
    pltpu.sync_copy(hn2_hbm, hn2_v)
    pltpu.sync_copy(src_hbm.at[pl.ds(base, EW)], sidx_v)
    pltpu.sync_copy(dst_hbm.at[pl.ds(base, EW)], didx_v)

    lanes = lax.iota(jnp.int32, 16)
    bufs = ((rows0_v, cidx0_v, sem0), (rows1_v, cidx1_v, sem1))

    def issue(k, p):
        rows_v, cidx_v, sem = bufs[p]
        off = k * BA
        for g in range(BA // 16):
            s16 = sidx_v[pl.ds(off + g * 16, 16)]
            d16 = didx_v[pl.ds(off + g * 16, 16)]
            pos = 2 * (g * 16 + lanes)
            plsc.store_scatter(cidx_v, [pos], s16)
            plsc.store_scatter(cidx_v, [pos + 1], d16)
        pltpu.async_copy(h_hbm.at[cidx_v], rows_v, sem)

    def process(k, p):
        rows_v, cidx_v, sem = bufs[p]
        pltpu.make_async_copy(h_hbm.at[cidx_v], rows_v, sem).wait()
        off = k * BA
        for g in range(BA // 16):
            def epart(i, c):
                e = g * 16 + i
                acc = rows_v[2 * e, pl.ds(0, 16)] * rows_v[2 * e + 1, pl.ds(0, 16)]
                for j in range(1, D // 16):
                    acc = acc + (rows_v[2 * e, pl.ds(j * 16, 16)]
                                 * rows_v[2 * e + 1, pl.ds(j * 16, 16)])
                plsc.store_scatter(xyt_v, [lanes * 16 + i], acc)
                return c

            lax.fori_loop(0, 16, epart, 0)
            xy = xyt_v[pl.ds(0, 16)]
            for l in range(1, 16):
                xy = xy + xyt_v[pl.ds(l * 16, 16)]
            s16 = sidx_v[pl.ds(off + g * 16, 16)]
            d16 = didx_v[pl.ds(off + g * 16, 16)]
            x2 = plsc.load_gather(hn2_v, [s16])
            y2 = plsc.load_gather(hn2_v, [d16])
            a = 1.0 - 2.0 * xy + y2
            b = 1.0 - x2
            den = 1.0 - 2.0 * xy + x2 * y2
            den = jnp.maximum(den, 1e-15)
            nm2b_v[pl.ds(off + g * 16, 16)] = (
                (a * a * x2 - 2.0 * a * b * xy + b * b * y2) / (den * den))

    issue(0, 0)

    def pair(t, carry):
        k0 = 2 * t
        issue(k0 + 1, 1)
        process(k0, 0)

        @pl.when(t < NCA // 2 - 1)
        def _():
            issue(k0 + 2, 0)

        process(k0 + 1, 1)
        return carry

    lax.fori_loop(0, NCA // 2, pair, 0)
    process_tail = NCA % 2
    if process_tail:
        issue(NCA - 1, 0)
        process(NCA - 1, 0)
    pltpu.sync_copy(nm2b_v, nm2_hbm.at[pl.ds(base, EW)])


def _sca(h, src, dst, hn2):
    mesh = plsc.VectorSubcoreMesh(core_axis_name="c", subcore_axis_name="s")
    f = pl.kernel(
        _sca_body,
        out_type=jax.ShapeDtypeStruct((E,), jnp.float32),
        mesh=mesh,
        compiler_params=pltpu.CompilerParams(needs_layout_passes=False),
        scratch_types=[
            pltpu.VMEM((N,), jnp.float32),
            pltpu.VMEM((EW,), jnp.int32),
            pltpu.VMEM((EW,), jnp.int32),
            pltpu.VMEM((EW,), jnp.float32),
            pltpu.VMEM((2 * BA, D), jnp.float32),
            pltpu.VMEM((2 * BA, D), jnp.float32),
            pltpu.VMEM((2 * BA,), jnp.int32),
            pltpu.VMEM((2 * BA,), jnp.int32),
            pltpu.VMEM((256,), jnp.float32),
            pltpu.SemaphoreType.DMA,
            pltpu.SemaphoreType.DMA,
        ],
    )
    return f(h, src, dst, hn2)


def _tcb_body(nm2_ref, ah_ref, w_ref):
    n = jnp.sqrt(jnp.clip(nm2_ref[...], 1e-15, None))
    pd = 2.0 * _artanh(n)
    z = -ah_ref[0, 0] * pd
    z = jnp.where(z >= 0.0, z, 0.2 * z)
    w_ref[...] = jnp.exp(z)


def _tcb(nm2, a_h):
    RE = E // D
    return pl.pallas_call(
        _tcb_body,
        in_specs=[
            pl.BlockSpec((RE, D), lambda: (0, 0)),
            pl.BlockSpec(memory_space=pltpu.SMEM),
        ],
        out_specs=pl.BlockSpec((RE, D), lambda: (0, 0)),
        out_shape=jax.ShapeDtypeStruct((RE, D), jnp.float32),
    )(nm2.reshape(RE, D), a_h.reshape(1, 1))


def _scc_body(ht_hbm, he_hbm, wh_hbm, src_hbm, dst_hbm, sa_hbm, sb_hbm,
              acch_hbm, dh_hbm, acce_hbm, de_hbm,
              acc_sh, accd_sh, auxa_v, auxb_v,
              sidx_vs, didx_vs, w_vs, rows_vs, isems, gsems):
    cid = lax.axis_index("c")
    sid = lax.axis_index("s")
    ebase = sid * ET

    def zzero(r, c):
        for j in range(D // 16):
            rows_vs[0][r, pl.ds(j * 16, 16)] = jnp.zeros((16,), jnp.float32)
        return c

    lax.fori_loop(0, BC, zzero, 0)
    for g in range(BC // 16):
        w_vs[0][pl.ds(g * 16, 16)] = jnp.zeros((16,), jnp.float32)

    @pl.when(sid < NZT)
    def _():
        r0 = sid * ZR
        for q in range(ZR // BC):
            pltpu.sync_copy(rows_vs[0], acc_sh.at[pl.ds(r0 + q * BC, BC)])
            pltpu.sync_copy(w_vs[0], accd_sh.at[pl.ds(r0 + q * BC, BC)])
        rem = ZR % BC
        if rem:
            q0 = r0 + (ZR // BC) * BC
            pltpu.sync_copy(rows_vs[0].at[pl.ds(0, rem)],
                            acc_sh.at[pl.ds(q0, rem)])
            pltpu.sync_copy(w_vs[0].at[pl.ds(0, rem)],
                            accd_sh.at[pl.ds(q0, rem)])

    @pl.when(cid == 1)
    def _():
        pltpu.sync_copy(sa_hbm, auxa_v)
        pltpu.sync_copy(sb_hbm, auxb_v)

    plsc.subcore_barrier()

    def mk_idx_copies(c, q, compute_w):
        off = ebase + c * BC
        cps = [
            pltpu.make_async_copy(src_hbm.at[pl.ds(off, BC)], sidx_vs[q],
                                  isems[q]),
            pltpu.make_async_copy(dst_hbm.at[pl.ds(off, BC)], didx_vs[q],
                                  isems[q]),
        ]
        if not compute_w:
            cps.append(pltpu.make_async_copy(wh_hbm.at[pl.ds(off, BC)],
                                             w_vs[q], isems[q]))
        return cps

    def issue_idx(c, q, compute_w):
        for cp in mk_idx_copies(c, q, compute_w):
            cp.start()

    def wait_idx(c, q, compute_w):
        for cp in mk_idx_copies(c, q, compute_w):
            cp.wait()

    def issue_gather(q, p, rows_hbm):
        pltpu.async_copy(rows_hbm.at[sidx_vs[q]], rows_vs[p], gsems[p])

    def wait_scatter(q, p):
        pltpu.make_async_copy(rows_vs[p], acc_sh.at[didx_vs[q]],
                              gsems[p]).wait()
        pltpu.make_async_copy(w_vs[q], accd_sh.at[didx_vs[q]],
                              gsems[p]).wait()

    def process(q, p, rows_hbm, compute_w):
        rows_v = rows_vs[p]
        w_s = w_vs[q] if compute_w else w_vs[q]
        if compute_w:
            for g in range(BC // 16):
                s16 = sidx_vs[q][pl.ds(g * 16, 16)]
                d16 = didx_vs[q][pl.ds(g * 16, 16)]
                lg = (plsc.load_gather(auxa_v, [s16])
                      + plsc.load_gather(auxb_v, [d16]))
                lg = jnp.where(lg >= 0.0, lg, 0.2 * lg)
                w_s[pl.ds(g * 16, 16)] = jnp.exp(lg)
        pltpu.make_async_copy(rows_hbm.at[sidx_vs[q]], rows_v,
                              gsems[p]).wait()

        def scale(t, c):
            for u in range(4):
                e = 4 * t + u
                wv = plsc.load_gather(w_s, [jnp.zeros((16,), jnp.int32) + e])
                for j in range(D // 16):
                    rows_v[e, pl.ds(j * 16, 16)] = (
                        rows_v[e, pl.ds(j * 16, 16)] * wv)
            return c

        lax.fori_loop(0, BC // 4, scale, 0)
        pltpu.async_copy(rows_v, acc_sh.at[didx_vs[q]], gsems[p], add=True)
        pltpu.async_copy(w_s, accd_sh.at[didx_vs[q]], gsems[p], add=True)

    def run(rows_hbm, compute_w):
        issue_idx(0, 0, compute_w)
        issue_idx(1, 1, compute_w)
        wait_idx(0, 0, compute_w)
        issue_gather(0, 0, rows_hbm)

        def step(c, i):
            q, p = i % 4, i % 2
            qn, pn = (i + 1) % 4, (i + 1) % 2

            @pl.when(c >= 1)
            def _():
                wait_scatter((i - 1) % 4, (i - 1) % 2)

            @pl.when(c + 1 < NCC)
            def _():
                wait_idx(c + 1, qn, compute_w)
                issue_gather(qn, pn, rows_hbm)

            @pl.when(c + 2 < NCC)
            def _():
                issue_idx(c + 2, (i + 2) % 4, compute_w)

            process(q, p, rows_hbm, compute_w)

        def quad(t, carry):
            c0 = 4 * t
            for i in range(4):
                step(c0 + i, i)
            return carry

        lax.fori_loop(0, NCC // 4, quad, 0)
        for i in range(NCC % 4):
            step((NCC // 4) * 4 + i, i)
        wait_scatter((NCC - 1) % 4, (NCC - 1) % 2)

    @pl.when(cid == 0)
    def _():
        run(ht_hbm, False)

    @pl.when(cid == 1)
    def _():
        run(he_hbm, True)

    plsc.subcore_barrier()

    @pl.when(sid < NZT)
    def _():
        r0 = sid * ZR

        def wb(vals_hbm, d_hbm):
            pltpu.sync_copy(acc_sh.at[pl.ds(r0, ZR)], vals_hbm.at[pl.ds(r0, ZR)])
            for q in range(ZR // BC):
                pltpu.sync_copy(accd_sh.at[pl.ds(r0 + q * BC, BC)], w_vs[0])
                pltpu.sync_copy(w_vs[0], d_hbm.at[pl.ds(r0 + q * BC, BC)])
            rem = ZR % BC
            if rem:
                q0 = r0 + (ZR // BC) * BC
                pltpu.sync_copy(accd_sh.at[pl.ds(q0, rem)],
                                w_vs[0].at[pl.ds(0, rem)])
                pltpu.sync_copy(w_vs[0].at[pl.ds(0, rem)],
                                d_hbm.at[pl.ds(q0, rem)])

        @pl.when(cid == 0)
        def _():
            wb(acch_hbm, dh_hbm)

        @pl.when(cid == 1)
        def _():
            wb(acce_hbm, de_hbm)


def _scc(ht, he, wh, src, dst, sa, sb):
    mesh = plsc.VectorSubcoreMesh(core_axis_name="c", subcore_axis_name="s")
    f = pl.kernel(
        _scc_body,
        out_type=[
            jax.ShapeDtypeStruct((N, D), jnp.float32),
            jax.ShapeDtypeStruct((N,), jnp.float32),
            jax.ShapeDtypeStruct((N, D), jnp.float32),
            jax.ShapeDtypeStruct((N,), jnp.float32),
        ],
        mesh=mesh,
        compiler_params=pltpu.CompilerParams(needs_layout_passes=False),
        scratch_types=[
            pltpu.VMEM_SHARED((N, D), jnp.float32),
            pltpu.VMEM_SHARED((N,), jnp.float32),
            pltpu.VMEM((N,), jnp.float32),
            pltpu.VMEM((N,), jnp.float32),
            [pltpu.VMEM((BC,), jnp.int32) for _ in range(4)],
            [pltpu.VMEM((BC,), jnp.int32) for _ in range(4)],
            [pltpu.VMEM((BC,), jnp.float32) for _ in range(4)],
            [pltpu.VMEM((BC, D), jnp.float32) for _ in range(2)],
            [pltpu.SemaphoreType.DMA for _ in range(4)],
            [pltpu.SemaphoreType.DMA for _ in range(2)],
        ],
    )
    return f(ht, he, wh, src, dst, sa, sb)


def _tcd_body(acch_ref, dh_ref, acce_ref, de_ref, athf_ref, atef_ref,
              xh_ref, xe_ref):
    agg_h = acch_ref[...] / (dh_ref[...] + 1e-16)
    x_h = _expmap0(jnp.maximum(agg_h, 0.0))
    xe2 = jnp.maximum(acce_ref[...] / (de_ref[...] + 1e-16), 0.0)
    xe_hyp = _expmap0(xe2)
    dist_f = _pdist(x_h, xe_hyp) * athf_ref[0, 0]
    nx = _norm(xe_hyp)
    ms = jnp.tanh(dist_f[:, None] * _artanh(nx)) * xe_hyp / nx
    x_h = _mobius_add(x_h, ms)
    xh_ref[...] = x_h
    log_xh = _logmap0(x_h)
    dist_e = jnp.sum((log_xh - xe2) ** 2, axis=-1, keepdims=True) * atef_ref[0, 0]
    xe_ref[...] = xe2 + dist_e * log_xh


def _tcd(acch, dh, acce, de, att_hf, att_ef):
    R = 1000
    grid = N // R
    row = lambda i: (i, 0)
    return pl.pallas_call(
        _tcd_body,
        grid=(grid,),
        in_specs=[
            pl.BlockSpec((R, D), row),
            pl.BlockSpec((R, 1), row),
            pl.BlockSpec((R, D), row),
            pl.BlockSpec((R, 1), row),
            pl.BlockSpec(memory_space=pltpu.SMEM),
            pl.BlockSpec(memory_space=pltpu.SMEM),
        ],
        out_specs=[
            pl.BlockSpec((R, D), row),
            pl.BlockSpec((R, D), row),
        ],
        out_shape=[
            jax.ShapeDtypeStruct((N, D), jnp.float32),
            jax.ShapeDtypeStruct((N, D), jnp.float32),
        ],
    )(acch, dh, acce, de, att_hf.reshape(1, 1), att_ef.reshape(1, 1))


def kernel(x, x_e, edge_index, W_h, b_h, a_h, W_e, b_e, a_src, a_dst,
           att_hf, att_ef):
    src = edge_index[0]
    dst = edge_index[1]
    ht, h, he, hn2, sa, sb = _pre(x, x_e, W_h, b_h, W_e, b_e, a_src, a_dst)
    nm2 = _sca(h, src, dst, hn2.reshape(N))
    wh = _tcb(nm2, a_h).reshape(E)
    acch, dh, acce, de = _scc(ht, he, wh, src, dst,
                              sa.reshape(N), sb.reshape(N))
    return _tcd(acch, dh.reshape(N, 1), acce, de.reshape(N, 1),
                att_hf, att_ef)


# speedup vs baseline: 1.0695x; 1.0000x over previous
import jax
import jax.numpy as jnp
from jax import lax
from jax.experimental import pallas as pl
from jax.experimental.pallas import tpu as pltpu
from jax.experimental.pallas import tpu_sc as plsc

N = 10000
E = 320000
D = 128
_MAX = 1.0 - 1e-5

NC = 2
NS = 16
NW = NC * NS
EW = E // NW
BA = 80
NCA = EW // BA
ET = E // NS
BC = 80
NCC = ET // BC
ZR = 1000
NZT = N // ZR
ZB = 50


def _norm(x):
    return jnp.sqrt(jnp.clip(jnp.sum(x * x, axis=-1, keepdims=True), 1e-15, None))


def _artanh(z):
    z = jnp.clip(z, -_MAX, _MAX)
    return 0.5 * jnp.log((1.0 + z) / (1.0 - z))


def _mobius_add(x, y):
    x2 = jnp.sum(x * x, -1, keepdims=True)
    y2 = jnp.sum(y * y, -1, keepdims=True)
    xy = jnp.sum(x * y, -1, keepdims=True)
    num = (1.0 + 2.0 * xy + y2) * x + (1.0 - x2) * y
    den = 1.0 + 2.0 * xy + x2 * y2
    return num / jnp.clip(den, 1e-15, None)


def _pdist(x, y):
    return 2.0 * _artanh(jnp.squeeze(_norm(_mobius_add(-x, y)), -1))


def _expmap0(u):
    n = _norm(u)
    return jnp.tanh(n) * u / n


def _logmap0(x):
    n = _norm(x)
    return _artanh(n) * x / n


def _pre_body(x_ref, xe_ref, wh_ref, we_ref, bh_ref, be_ref, asrc_ref,
              adst_ref, ht_ref, h_ref, he_ref, hn2_ref, sa_ref, sb_ref):
    x = x_ref[...]
    u = _logmap0(x)
    ht = jnp.dot(u, wh_ref[...], preferred_element_type=jnp.float32) + bh_ref[...]
    ht_ref[...] = ht
    nt = _norm(ht)
    th = jnp.tanh(nt)
    h_ref[...] = th * ht / nt
    hn2_ref[...] = th * th
    he = jnp.dot(xe_ref[...], we_ref[...], preferred_element_type=jnp.float32) + be_ref[...]
    he_ref[...] = he
    sa_ref[...] = jnp.dot(he, asrc_ref[...], preferred_element_type=jnp.float32)
    sb_ref[...] = jnp.dot(he, adst_ref[...], preferred_element_type=jnp.float32)


def _pre(x, x_e, W_h, b_h, W_e, b_e, a_src, a_dst):
    R = 1000
    grid = N // R
    row = lambda i: (i, 0)
    fixed = lambda i: (0, 0)
    return pl.pallas_call(
        _pre_body,
        grid=(grid,),
        in_specs=[
            pl.BlockSpec((R, D), row),
            pl.BlockSpec((R, D), row),
            pl.BlockSpec((D, D), fixed),
            pl.BlockSpec((D, D), fixed),
            pl.BlockSpec((1, D), fixed),
            pl.BlockSpec((1, D), fixed),
            pl.BlockSpec((D, 1), fixed),
            pl.BlockSpec((D, 1), fixed),
        ],
        out_specs=[
            pl.BlockSpec((R, D), row),
            pl.BlockSpec((R, D), row),
            pl.BlockSpec((R, D), row),
            pl.BlockSpec((R, 1), row),
            pl.BlockSpec((R, 1), row),
            pl.BlockSpec((R, 1), row),
        ],
        out_shape=[
            jax.ShapeDtypeStruct((N, D), jnp.float32),
            jax.ShapeDtypeStruct((N, D), jnp.float32),
            jax.ShapeDtypeStruct((N, D), jnp.float32),
            jax.ShapeDtypeStruct((N, 1), jnp.float32),
            jax.ShapeDtypeStruct((N, 1), jnp.float32),
            jax.ShapeDtypeStruct((N, 1), jnp.float32),
        ],
    )(x, x_e, W_h, W_e, b_h.reshape(1, D), b_e.reshape(1, D),
      a_src.reshape(D, 1), a_dst.reshape(D, 1))


def _sca_body(h_hbm, src_hbm, dst_hbm, hn2_hbm, nm2_hbm,
              hn2_v, sidx_v, didx_v, nm2b_v, rows0_v, rows1_v,
              cidx0_v, cidx1_v, xyt_v, sem0, sem1):
    cid = lax.axis_index("c")
    sid = lax.axis_index("s")
    wid = sid * NC + cid
    base = wid * EW
    pltpu.sync_copy(hn2_hbm, hn2_v)
    pltpu.sync_copy(src_hbm.at[pl.ds(base, EW)], sidx_v)
    pltpu.sync_copy(dst_hbm.at[pl.ds(base, EW)], didx_v)

    lanes = lax.iota(jnp.int32, 16)
    bufs = ((rows0_v, cidx0_v, sem0), (rows1_v, cidx1_v, sem1))

    def issue(k, p):
        rows_v, cidx_v, sem = bufs[p]
        off = k * BA
        for g in range(BA // 16):
            s16 = sidx_v[pl.ds(off + g * 16, 16)]
            d16 = didx_v[pl.ds(off + g * 16, 16)]
            pos = 2 * (g * 16 + lanes)
            plsc.store_scatter(cidx_v, [pos], s16)
            plsc.store_scatter(cidx_v, [pos + 1], d16)
        pltpu.async_copy(h_hbm.at[cidx_v], rows_v, sem)

    def process(k, p):
        rows_v, cidx_v, sem = bufs[p]
        pltpu.make_async_copy(h_hbm.at[cidx_v], rows_v, sem).wait()
        off = k * BA
        for g in range(BA // 16):
            def epart(i, c):
                e = g * 16 + i
                acc = rows_v[2 * e, pl.ds(0, 16)] * rows_v[2 * e + 1, pl.ds(0, 16)]
                for j in range(1, D // 16):
                    acc = acc + (rows_v[2 * e, pl.ds(j * 16, 16)]
                                 * rows_v[2 * e + 1, pl.ds(j * 16, 16)])
                plsc.store_scatter(xyt_v, [lanes * 16 + i], acc)
                return c

            lax.fori_loop(0, 16, epart, 0)
            xy = xyt_v[pl.ds(0, 16)]
            for l in range(1, 16):
                xy = xy + xyt_v[pl.ds(l * 16, 16)]
            s16 = sidx_v[pl.ds(off + g * 16, 16)]
            d16 = didx_v[pl.ds(off + g * 16, 16)]
            x2 = plsc.load_gather(hn2_v, [s16])
            y2 = plsc.load_gather(hn2_v, [d16])
            a = 1.0 - 2.0 * xy + y2
            b = 1.0 - x2
            den = 1.0 - 2.0 * xy + x2 * y2
            den = jnp.maximum(den, 1e-15)
            nm2b_v[pl.ds(off + g * 16, 16)] = (
                (a * a * x2 - 2.0 * a * b * xy + b * b * y2) / (den * den))

    issue(0, 0)

    def pair(t, carry):
        k0 = 2 * t
        issue(k0 + 1, 1)
        process(k0, 0)

        @pl.when(t < NCA // 2 - 1)
        def _():
            issue(k0 + 2, 0)

        process(k0 + 1, 1)
        return carry

    lax.fori_loop(0, NCA // 2, pair, 0)
    process_tail = NCA % 2
    if process_tail:
        issue(NCA - 1, 0)
        process(NCA - 1, 0)
    pltpu.sync_copy(nm2b_v, nm2_hbm.at[pl.ds(base, EW)])


def _sca(h, src, dst, hn2):
    mesh = plsc.VectorSubcoreMesh(core_axis_name="c", subcore_axis_name="s")
    f = pl.kernel(
        _sca_body,
        out_type=jax.ShapeDtypeStruct((E,), jnp.float32),
        mesh=mesh,
        compiler_params=pltpu.CompilerParams(needs_layout_passes=False),
        scratch_types=[
            pltpu.VMEM((N,), jnp.float32),
            pltpu.VMEM((EW,), jnp.int32),
            pltpu.VMEM((EW,), jnp.int32),
            pltpu.VMEM((EW,), jnp.float32),
            pltpu.VMEM((2 * BA, D), jnp.float32),
            pltpu.VMEM((2 * BA, D), jnp.float32),
            pltpu.VMEM((2 * BA,), jnp.int32),
            pltpu.VMEM((2 * BA,), jnp.int32),
            pltpu.VMEM((256,), jnp.float32),
            pltpu.SemaphoreType.DMA,
            pltpu.SemaphoreType.DMA,
        ],
    )
    return f(h, src, dst, hn2)


def _tcb_body(nm2_ref, ah_ref, w_ref):
    n = jnp.sqrt(jnp.clip(nm2_ref[...], 1e-15, None))
    pd = 2.0 * _artanh(n)
    z = -ah_ref[0, 0] * pd
    z = jnp.where(z >= 0.0, z, 0.2 * z)
    w_ref[...] = jnp.exp(z)


def _tcb(nm2, a_h):
    RE = E // D
    return pl.pallas_call(
        _tcb_body,
        in_specs=[
            pl.BlockSpec((RE, D), lambda: (0, 0)),
            pl.BlockSpec(memory_space=pltpu.SMEM),
        ],
        out_specs=pl.BlockSpec((RE, D), lambda: (0, 0)),
        out_shape=jax.ShapeDtypeStruct((RE, D), jnp.float32),
    )(nm2.reshape(RE, D), a_h.reshape(1, 1))


def _scc_body(ht_hbm, he_hbm, wh_hbm, src_hbm, dst_hbm, sa_hbm, sb_hbm,
              acch_hbm, dh_hbm, acce_hbm, de_hbm,
              acc_sh, accd_sh, auxa_v, auxb_v,
              sidx_vs, didx_vs, w_vs, rows_vs, isems, gsems):
    cid = lax.axis_index("c")
    sid = lax.axis_index("s")
    ebase = sid * ET

    def zzero(r, c):
        for j in range(D // 16):
            rows_vs[0][r, pl.ds(j * 16, 16)] = jnp.zeros((16,), jnp.float32)
        return c

    lax.fori_loop(0, BC, zzero, 0)
    for g in range(BC // 16):
        w_vs[0][pl.ds(g * 16, 16)] = jnp.zeros((16,), jnp.float32)

    @pl.when(sid < NZT)
    def _():
        r0 = sid * ZR
        for q in range(ZR // BC):
            pltpu.sync_copy(rows_vs[0], acc_sh.at[pl.ds(r0 + q * BC, BC)])
            pltpu.sync_copy(w_vs[0], accd_sh.at[pl.ds(r0 + q * BC, BC)])
        rem = ZR % BC
        if rem:
            q0 = r0 + (ZR // BC) * BC
            pltpu.sync_copy(rows_vs[0].at[pl.ds(0, rem)],
                            acc_sh.at[pl.ds(q0, rem)])
            pltpu.sync_copy(w_vs[0].at[pl.ds(0, rem)],
                            accd_sh.at[pl.ds(q0, rem)])

    @pl.when(cid == 1)
    def _():
        pltpu.sync_copy(sa_hbm, auxa_v)
        pltpu.sync_copy(sb_hbm, auxb_v)

    plsc.subcore_barrier()

    def mk_idx_copies(c, q, compute_w):
        off = ebase + c * BC
        cps = [
            pltpu.make_async_copy(src_hbm.at[pl.ds(off, BC)], sidx_vs[q],
                                  isems[q]),
            pltpu.make_async_copy(dst_hbm.at[pl.ds(off, BC)], didx_vs[q],
                                  isems[q]),
        ]
        if not compute_w:
            cps.append(pltpu.make_async_copy(wh_hbm.at[pl.ds(off, BC)],
                                             w_vs[q], isems[q]))
        return cps

    def issue_idx(c, q, compute_w):
        for cp in mk_idx_copies(c, q, compute_w):
            cp.start()

    def wait_idx(c, q, compute_w):
        for cp in mk_idx_copies(c, q, compute_w):
            cp.wait()

    def issue_gather(q, p, rows_hbm):
        pltpu.async_copy(rows_hbm.at[sidx_vs[q]], rows_vs[p], gsems[p])

    def wait_scatter(q, p):
        pltpu.make_async_copy(rows_vs[p], acc_sh.at[didx_vs[q]],
                              gsems[p]).wait()
        pltpu.make_async_copy(w_vs[q], accd_sh.at[didx_vs[q]],
                              gsems[p]).wait()

    def process(q, p, rows_hbm, compute_w):
        rows_v = rows_vs[p]
        w_s = w_vs[q] if compute_w else w_vs[q]
        if compute_w:
            for g in range(BC // 16):
                s16 = sidx_vs[q][pl.ds(g * 16, 16)]
                d16 = didx_vs[q][pl.ds(g * 16, 16)]
                lg = (plsc.load_gather(auxa_v, [s16])
                      + plsc.load_gather(auxb_v, [d16]))
                lg = jnp.where(lg >= 0.0, lg, 0.2 * lg)
                w_s[pl.ds(g * 16, 16)] = jnp.exp(lg)
        pltpu.make_async_copy(rows_hbm.at[sidx_vs[q]], rows_v,
                              gsems[p]).wait()

        def scale(t, c):
            for u in range(4):
                e = 4 * t + u
                wv = plsc.load_gather(w_s, [jnp.zeros((16,), jnp.int32) + e])
                for j in range(D // 16):
                    rows_v[e, pl.ds(j * 16, 16)] = (
                        rows_v[e, pl.ds(j * 16, 16)] * wv)
            return c

        lax.fori_loop(0, BC // 4, scale, 0)
        pltpu.async_copy(rows_v, acc_sh.at[didx_vs[q]], gsems[p], add=True)
        pltpu.async_copy(w_s, accd_sh.at[didx_vs[q]], gsems[p], add=True)

    def run(rows_hbm, compute_w):
        issue_idx(0, 0, compute_w)
        issue_idx(1, 1, compute_w)
        wait_idx(0, 0, compute_w)
        issue_gather(0, 0, rows_hbm)

        def step(c, i):
            q, p = i % 4, i % 2
            qn, pn = (i + 1) % 4, (i + 1) % 2

            @pl.when(c >= 1)
            def _():
                wait_scatter((i - 1) % 4, (i - 1) % 2)

            @pl.when(c + 1 < NCC)
            def _():
                wait_idx(c + 1, qn, compute_w)
                issue_gather(qn, pn, rows_hbm)

            @pl.when(c + 2 < NCC)
            def _():
                issue_idx(c + 2, (i + 2) % 4, compute_w)

            process(q, p, rows_hbm, compute_w)

        def quad(t, carry):
            c0 = 4 * t
            for i in range(4):
                step(c0 + i, i)
            return carry

        lax.fori_loop(0, NCC // 4, quad, 0)
        for i in range(NCC % 4):
            step((NCC // 4) * 4 + i, i)
        wait_scatter((NCC - 1) % 4, (NCC - 1) % 2)

    @pl.when(cid == 0)
    def _():
        run(ht_hbm, False)

    @pl.when(cid == 1)
    def _():
        run(he_hbm, True)

    plsc.subcore_barrier()

    @pl.when(sid < NZT)
    def _():
        r0 = sid * ZR

        def wb(vals_hbm, d_hbm):
            pltpu.sync_copy(acc_sh.at[pl.ds(r0, ZR)], vals_hbm.at[pl.ds(r0, ZR)])
            for q in range(ZR // BC):
                pltpu.sync_copy(accd_sh.at[pl.ds(r0 + q * BC, BC)], w_vs[0])
                pltpu.sync_copy(w_vs[0], d_hbm.at[pl.ds(r0 + q * BC, BC)])
            rem = ZR % BC
            if rem:
                q0 = r0 + (ZR // BC) * BC
                pltpu.sync_copy(accd_sh.at[pl.ds(q0, rem)],
                                w_vs[0].at[pl.ds(0, rem)])
                pltpu.sync_copy(w_vs[0].at[pl.ds(0, rem)],
                                d_hbm.at[pl.ds(q0, rem)])

        @pl.when(cid == 0)
        def _():
            wb(acch_hbm, dh_hbm)

        @pl.when(cid == 1)
        def _():
            wb(acce_hbm, de_hbm)


def _scc(ht, he, wh, src, dst, sa, sb):
    mesh = plsc.VectorSubcoreMesh(core_axis_name="c", subcore_axis_name="s")
    f = pl.kernel(
        _scc_body,
        out_type=[
            jax.ShapeDtypeStruct((N, D), jnp.float32),
            jax.ShapeDtypeStruct((N,), jnp.float32),
            jax.ShapeDtypeStruct((N, D), jnp.float32),
            jax.ShapeDtypeStruct((N,), jnp.float32),
        ],
        mesh=mesh,
        compiler_params=pltpu.CompilerParams(needs_layout_passes=False),
        scratch_types=[
            pltpu.VMEM_SHARED((N, D), jnp.float32),
            pltpu.VMEM_SHARED((N,), jnp.float32),
            pltpu.VMEM((N,), jnp.float32),
            pltpu.VMEM((N,), jnp.float32),
            [pltpu.VMEM((BC,), jnp.int32) for _ in range(4)],
            [pltpu.VMEM((BC,), jnp.int32) for _ in range(4)],
            [pltpu.VMEM((BC,), jnp.float32) for _ in range(4)],
            [pltpu.VMEM((BC, D), jnp.float32) for _ in range(2)],
            [pltpu.SemaphoreType.DMA for _ in range(4)],
            [pltpu.SemaphoreType.DMA for _ in range(2)],
        ],
    )
    return f(ht, he, wh, src, dst, sa, sb)


def _tcd_body(acch_ref, dh_ref, acce_ref, de_ref, athf_ref, atef_ref,
              xh_ref, xe_ref):
    agg_h = acch_ref[...] / (dh_ref[...] + 1e-16)
    x_h = _expmap0(jnp.maximum(agg_h, 0.0))
    xe2 = jnp.maximum(acce_ref[...] / (de_ref[...] + 1e-16), 0.0)
    xe_hyp = _expmap0(xe2)
    dist_f = _pdist(x_h, xe_hyp) * athf_ref[0, 0]
    nx = _norm(xe_hyp)
    ms = jnp.tanh(dist_f[:, None] * _artanh(nx)) * xe_hyp / nx
    x_h = _mobius_add(x_h, ms)
    xh_ref[...] = x_h
    log_xh = _logmap0(x_h)
    dist_e = jnp.sum((log_xh - xe2) ** 2, axis=-1, keepdims=True) * atef_ref[0, 0]
    xe_ref[...] = xe2 + dist_e * log_xh


def _tcd(acch, dh, acce, de, att_hf, att_ef):
    R = 1000
    grid = N // R
    row = lambda i: (i, 0)
    return pl.pallas_call(
        _tcd_body,
        grid=(grid,),
        in_specs=[
            pl.BlockSpec((R, D), row),
            pl.BlockSpec((R, 1), row),
            pl.BlockSpec((R, D), row),
            pl.BlockSpec((R, 1), row),
            pl.BlockSpec(memory_space=pltpu.SMEM),
            pl.BlockSpec(memory_space=pltpu.SMEM),
        ],
        out_specs=[
            pl.BlockSpec((R, D), row),
            pl.BlockSpec((R, D), row),
        ],
        out_shape=[
            jax.ShapeDtypeStruct((N, D), jnp.float32),
            jax.ShapeDtypeStruct((N, D), jnp.float32),
        ],
    )(acch, dh, acce, de, att_hf.reshape(1, 1), att_ef.reshape(1, 1))


def kernel(x, x_e, edge_index, W_h, b_h, a_h, W_e, b_e, a_src, a_dst,
           att_hf, att_ef):
    src = edge_index[0]
    dst = edge_index[1]
    ht, h, he, hn2, sa, sb = _pre(x, x_e, W_h, b_h, W_e, b_e, a_src, a_dst)
    nm2 = _sca(h, src, dst, hn2.reshape(N))
    wh = _tcb(nm2, a_h).reshape(E)
    acch, dh, acce, de = _scc(ht, he, wh, src, dst,
                              sa.reshape(N), sb.reshape(N))
    return _tcd(acch, dh.reshape(N, 1), acce, de.reshape(N, 1),
                att_hf, att_ef)
